# Initial kernel scaffold; baseline (speedup 1.0000x reference)
import jax
import jax.numpy as jnp
from jax.experimental import pallas as pl


def kernel(x, edge_index, W, b):
    raise NotImplementedError("write your pallas kernel here")


# speedup vs baseline: 13.4737x; 13.4737x over previous
import functools

import jax
import jax.numpy as jnp
from jax import lax
from jax.experimental import pallas as pl
from jax.experimental.pallas import tpu as pltpu, tpu_sc as plsc

NC = 2
NS = 16
CH = 128


def _deg_kernel(Ns, cpt, rpt):
    mesh = plsc.VectorSubcoreMesh(core_axis_name="c", subcore_axis_name="s",
                                  num_cores=NC, num_subcores=NS)

    @functools.partial(
        pl.kernel,
        out_type=jax.ShapeDtypeStruct((NC, Ns, 128), jnp.float32),
        mesh=mesh,
        scratch_types=[
            pltpu.VMEM((CH,), jnp.int32),
            pltpu.VMEM((CH, 128), jnp.float32),
            pltpu.VMEM_SHARED((Ns, 128), jnp.float32),
        ],
    )
    def k(rows_hbm, ones_hbm, zeros_hbm, out_hbm, rowv, srcv, deg_sh):
        c = lax.axis_index("c")
        s = lax.axis_index("s")
        wid = s * NC + c
        pltpu.sync_copy(zeros_hbm, deg_sh.at[pl.ds(s * rpt, rpt)])
        pltpu.sync_copy(ones_hbm, srcv)
        plsc.subcore_barrier()

        def body(j, carry):
            pltpu.sync_copy(rows_hbm.at[wid * cpt + j], rowv)
            pltpu.sync_copy(srcv, deg_sh.at[rowv], add=True)
            return carry

        lax.fori_loop(0, cpt, body, 0)
        plsc.subcore_barrier()
        pltpu.sync_copy(deg_sh.at[pl.ds(s * rpt, rpt)],
                        out_hbm.at[c, pl.ds(s * rpt, rpt)])

    return k


def _scatter_kernel(N, D, Ns, cpt, rpt):
    mesh = plsc.VectorSubcoreMesh(core_axis_name="c", subcore_axis_name="s",
                                  num_cores=NC, num_subcores=NS)

    @functools.partial(
        pl.kernel,
        out_type=jax.ShapeDtypeStruct((NC, Ns, D), jnp.float32),
        mesh=mesh,
        scratch_types=[
            pltpu.VMEM((CH,), jnp.int32),
            pltpu.VMEM((CH,), jnp.int32),
            pltpu.VMEM((CH, D), jnp.float32),
            pltpu.VMEM_SHARED((Ns, D), jnp.float32),
            pltpu.SemaphoreType.DMA,
        ],
    )
    def k(cols_hbm, rows_hbm, g_hbm, zeros_hbm, out_hbm,
          colv, rowv, gbuf, s_sh, sem):
        c = lax.axis_index("c")
        s = lax.axis_index("s")
        wid = s * NC + c
        pltpu.sync_copy(zeros_hbm, s_sh.at[pl.ds(s * rpt, rpt)])
        plsc.subcore_barrier()

        def body(j, carry):
            pltpu.sync_copy(cols_hbm.at[wid * cpt + j], colv)
            pltpu.async_copy(g_hbm.at[colv], gbuf, sem).wait()
            pltpu.sync_copy(rows_hbm.at[wid * cpt + j], rowv)
            pltpu.sync_copy(gbuf, s_sh.at[rowv], add=True)
            return carry

        lax.fori_loop(0, cpt, body, 0)
        plsc.subcore_barrier()
        pltpu.sync_copy(s_sh.at[pl.ds(s * rpt, rpt)],
                        out_hbm.at[c, pl.ds(s * rpt, rpt)])

    return k


def _linear_kernel(x_ref, w_ref, b_ref, p0_ref, p1_ref, g_ref):
    d = p0_ref[:, 0:1] + p1_ref[:, 0:1] + 1.0
    dsq = lax.rsqrt(d)
    h = lax.dot_general(x_ref[...], w_ref[...],
                        (((1,), (1,)), ((), ())),
                        preferred_element_type=jnp.float32) + b_ref[...]
    g_ref[...] = h * dsq


def _final_kernel(g_ref, s0_ref, s1_ref, p0_ref, p1_ref, o_ref):
    d = p0_ref[:, 0:1] + p1_ref[:, 0:1] + 1.0
    dsq = lax.rsqrt(d)
    o_ref[...] = dsq * (g_ref[...] + s0_ref[...] + s1_ref[...])


def kernel(x, edge_index, W, b):
    N, Din = x.shape
    Dout = W.shape[0]
    E = edge_index.shape[1]
    NW = NC * NS

    rows = edge_index[0].astype(jnp.int32)
    cols = edge_index[1].astype(jnp.int32)
    n_chunks = -(-E // CH)
    cpt = -(-n_chunks // NW)
    Cpad = cpt * NW
    Epad = Cpad * CH
    Ns = 8 * NS * (-(-(N + 1) // (8 * NS)))
    rpt = Ns // NS
    pad = Epad - E
    rows_p = jnp.concatenate(
        [rows, jnp.full((pad,), N, jnp.int32)]).reshape(Cpad, CH)
    cols_p = jnp.concatenate(
        [cols, jnp.zeros((pad,), jnp.int32)]).reshape(Cpad, CH)
    ones0 = jnp.zeros((CH, 128), jnp.float32).at[:, 0].set(1.0)
    zerosD = jnp.zeros((rpt, Dout), jnp.float32)
    b2 = b.reshape(1, Dout)

    degp = _deg_kernel(Ns, cpt, rpt)(rows_p, ones0, zerosD)
    p0, p1 = degp[0], degp[1]

    NB = 400
    nblk = -(-N // NB)
    g = pl.pallas_call(
        _linear_kernel,
        grid=(nblk,),
        in_specs=[
            pl.BlockSpec((NB, Din), lambda i: (i, 0)),
            pl.BlockSpec((Dout, Din), lambda i: (0, 0)),
            pl.BlockSpec((1, Dout), lambda i: (0, 0)),
            pl.BlockSpec((NB, 128), lambda i: (i, 0)),
            pl.BlockSpec((NB, 128), lambda i: (i, 0)),
        ],
        out_specs=pl.BlockSpec((NB, Dout), lambda i: (i, 0)),
        out_shape=jax.ShapeDtypeStruct((N, Dout), jnp.float32),
    )(x, W, b2, p0, p1)

    sp = _scatter_kernel(N, Dout, Ns, cpt, rpt)(cols_p, rows_p, g, zerosD)

    out = pl.pallas_call(
        _final_kernel,
        grid=(nblk,),
        in_specs=[
            pl.BlockSpec((NB, Dout), lambda i: (i, 0)),
            pl.BlockSpec((NB, Dout), lambda i: (i, 0)),
            pl.BlockSpec((NB, Dout), lambda i: (i, 0)),
            pl.BlockSpec((NB, 128), lambda i: (i, 0)),
            pl.BlockSpec((NB, 128), lambda i: (i, 0)),
        ],
        out_specs=pl.BlockSpec((NB, Dout), lambda i: (i, 0)),
        out_shape=jax.ShapeDtypeStruct((N, Dout), jnp.float32),
    )(g, sp[0], sp[1], p0, p1)
    return out


# speedup vs baseline: 13.5073x; 1.0025x over previous
import functools

import jax
import jax.numpy as jnp
from jax import lax
from jax.experimental import pallas as pl
from jax.experimental.pallas import tpu as pltpu, tpu_sc as plsc

NC = 2
NS = 16
CH = 128


def _deg_kernel(Ns, cpt, rpt):
    mesh = plsc.VectorSubcoreMesh(core_axis_name="c", subcore_axis_name="s",
                                  num_cores=NC, num_subcores=NS)

    @functools.partial(
        pl.kernel,
        out_type=jax.ShapeDtypeStruct((NC, Ns, 128), jnp.float32),
        mesh=mesh,
        scratch_types=[
            pltpu.VMEM((cpt, CH), jnp.int32),
            pltpu.VMEM((CH, 128), jnp.float32),
            pltpu.VMEM_SHARED((Ns, 128), jnp.float32),
        ],
    )
    def k(rows_hbm, ones_hbm, zeros_hbm, out_hbm, rowbuf, srcv, deg_sh):
        c = lax.axis_index("c")
        s = lax.axis_index("s")
        wid = s * NC + c
        pltpu.sync_copy(zeros_hbm, deg_sh.at[pl.ds(s * rpt, rpt)])
        pltpu.sync_copy(ones_hbm, srcv)
        pltpu.sync_copy(rows_hbm.at[wid], rowbuf)
        plsc.subcore_barrier()

        def body(j, carry):
            pltpu.sync_copy(srcv, deg_sh.at[rowbuf.at[j]], add=True)
            return carry

        lax.fori_loop(0, cpt, body, 0)
        plsc.subcore_barrier()
        pltpu.sync_copy(deg_sh.at[pl.ds(s * rpt, rpt)],
                        out_hbm.at[c, pl.ds(s * rpt, rpt)])

    return k


def _scatter_kernel(N, D, Ns, cpt, rpt):
    assert cpt % 2 == 0
    mesh = plsc.VectorSubcoreMesh(core_axis_name="c", subcore_axis_name="s",
                                  num_cores=NC, num_subcores=NS)

    @functools.partial(
        pl.kernel,
        out_type=jax.ShapeDtypeStruct((NC, Ns, D), jnp.float32),
        mesh=mesh,
        scratch_types=[
            pltpu.VMEM((cpt, CH), jnp.int32),
            pltpu.VMEM((CH,), jnp.int32),
            pltpu.VMEM((CH,), jnp.int32),
            pltpu.VMEM((CH,), jnp.int32),
            pltpu.VMEM((CH,), jnp.int32),
            pltpu.VMEM((CH, D), jnp.float32),
            pltpu.VMEM((CH, D), jnp.float32),
            pltpu.VMEM_SHARED((Ns, D), jnp.float32),
            pltpu.SemaphoreType.DMA,
            pltpu.SemaphoreType.DMA,
        ],
    )
    def k(packed_hbm, g_hbm, zeros_hbm, out_hbm,
          pbuf, colv0, rowv0, colv1, rowv1, gbuf0, gbuf1, s_sh, sem0, sem1):
        c = lax.axis_index("c")
        s = lax.axis_index("s")
        wid = s * NC + c
        pltpu.sync_copy(zeros_hbm, s_sh.at[pl.ds(s * rpt, rpt)])
        pltpu.sync_copy(packed_hbm.at[wid], pbuf)
        plsc.subcore_barrier()

        def unpack(j, colv, rowv):
            for kk in range(CH // 16):
                v = pbuf[j, pl.ds(kk * 16, 16)]
                colv[pl.ds(kk * 16, 16)] = lax.bitwise_and(v, 16383)
                rowv[pl.ds(kk * 16, 16)] = lax.shift_right_logical(v, 14)

        unpack(0, colv0, rowv0)
        pltpu.async_copy(g_hbm.at[colv0], gbuf0, sem0)
        unpack(1, colv1, rowv1)
        pltpu.async_copy(g_hbm.at[colv1], gbuf1, sem1)

        def body(p, carry):
            j = 2 * p
            pltpu.make_async_copy(g_hbm.at[colv0], gbuf0, sem0).wait()
            pltpu.sync_copy(gbuf0, s_sh.at[rowv0], add=True)

            @pl.when(j + 2 < cpt)
            def _():
                unpack(j + 2, colv0, rowv0)
                pltpu.async_copy(g_hbm.at[colv0], gbuf0, sem0)

            pltpu.make_async_copy(g_hbm.at[colv1], gbuf1, sem1).wait()
            pltpu.sync_copy(gbuf1, s_sh.at[rowv1], add=True)

            @pl.when(j + 3 < cpt)
            def _():
                unpack(j + 3, colv1, rowv1)
                pltpu.async_copy(g_hbm.at[colv1], gbuf1, sem1)

            return carry

        lax.fori_loop(0, cpt // 2, body, 0)
        plsc.subcore_barrier()
        pltpu.sync_copy(s_sh.at[pl.ds(s * rpt, rpt)],
                        out_hbm.at[c, pl.ds(s * rpt, rpt)])

    return k


def _linear_kernel(x_ref, w_ref, b_ref, p0_ref, p1_ref, g_ref):
    d = p0_ref[:, 0:1] + p1_ref[:, 0:1] + 1.0
    dsq = lax.rsqrt(d)
    h = lax.dot_general(x_ref[...], w_ref[...],
                        (((1,), (1,)), ((), ())),
                        preferred_element_type=jnp.float32) + b_ref[...]
    g_ref[...] = h * dsq


def _final_kernel(g_ref, s0_ref, s1_ref, p0_ref, p1_ref, o_ref):
    d = p0_ref[:, 0:1] + p1_ref[:, 0:1] + 1.0
    dsq = lax.rsqrt(d)
    o_ref[...] = dsq * (g_ref[...] + s0_ref[...] + s1_ref[...])


def kernel(x, edge_index, W, b):
    N, Din = x.shape
    Dout = W.shape[0]
    E = edge_index.shape[1]
    NW = NC * NS

    rows = edge_index[0].astype(jnp.int32)
    cols = edge_index[1].astype(jnp.int32)
    n_chunks = -(-E // CH)
    cpt = 2 * (-(-n_chunks // (2 * NW)))
    Cpad = cpt * NW
    Epad = Cpad * CH
    Ns = 8 * NS * (-(-(N + 1) // (8 * NS)))
    rpt = Ns // NS
    pad = Epad - E
    rows_p = jnp.concatenate(
        [rows, jnp.full((pad,), N, jnp.int32)]).reshape(cpt, NW, CH).transpose(1, 0, 2)
    cols_p = jnp.concatenate(
        [cols, jnp.zeros((pad,), jnp.int32)]).reshape(cpt, NW, CH).transpose(1, 0, 2)
    packed_p = rows_p * 16384 + cols_p
    ones0 = jnp.zeros((CH, 128), jnp.float32).at[:, 0].set(1.0)
    zerosD = jnp.zeros((rpt, Dout), jnp.float32)
    b2 = b.reshape(1, Dout)

    degp = _deg_kernel(Ns, cpt, rpt)(rows_p, ones0, zerosD)
    p0, p1 = degp[0], degp[1]

    NB = 400
    nblk = -(-N // NB)
    g = pl.pallas_call(
        _linear_kernel,
        grid=(nblk,),
        in_specs=[
            pl.BlockSpec((NB, Din), lambda i: (i, 0)),
            pl.BlockSpec((Dout, Din), lambda i: (0, 0)),
            pl.BlockSpec((1, Dout), lambda i: (0, 0)),
            pl.BlockSpec((NB, 128), lambda i: (i, 0)),
            pl.BlockSpec((NB, 128), lambda i: (i, 0)),
        ],
        out_specs=pl.BlockSpec((NB, Dout), lambda i: (i, 0)),
        out_shape=jax.ShapeDtypeStruct((N, Dout), jnp.float32),
    )(x, W, b2, p0, p1)

    sp = _scatter_kernel(N, Dout, Ns, cpt, rpt)(packed_p, g, zerosD)

    out = pl.pallas_call(
        _final_kernel,
        grid=(nblk,),
        in_specs=[
            pl.BlockSpec((NB, Dout), lambda i: (i, 0)),
            pl.BlockSpec((NB, Dout), lambda i: (i, 0)),
            pl.BlockSpec((NB, Dout), lambda i: (i, 0)),
            pl.BlockSpec((NB, 128), lambda i: (i, 0)),
            pl.BlockSpec((NB, 128), lambda i: (i, 0)),
        ],
        out_specs=pl.BlockSpec((NB, Dout), lambda i: (i, 0)),
        out_shape=jax.ShapeDtypeStruct((N, Dout), jnp.float32),
    )(g, sp[0], sp[1], p0, p1)
    return out


# speedup vs baseline: 13.8141x; 1.0227x over previous
import functools

import jax
import jax.numpy as jnp
from jax import lax
from jax.experimental import pallas as pl
from jax.experimental.pallas import tpu as pltpu, tpu_sc as plsc

NC = 2
NS = 16
CH = 128
EC = 64
NBUF = 4


def _deg_kernel(Ns, cpt, rpt):
    mesh = plsc.VectorSubcoreMesh(core_axis_name="c", subcore_axis_name="s",
                                  num_cores=NC, num_subcores=NS)

    @functools.partial(
        pl.kernel,
        out_type=jax.ShapeDtypeStruct((NC, Ns, 128), jnp.float32),
        mesh=mesh,
        scratch_types=[
            pltpu.VMEM((cpt, CH), jnp.int32),
            pltpu.VMEM((CH, 128), jnp.float32),
            pltpu.VMEM_SHARED((Ns, 128), jnp.float32),
        ],
    )
    def k(rows_hbm, ones_hbm, zeros_hbm, out_hbm, rowbuf, srcv, deg_sh):
        c = lax.axis_index("c")
        s = lax.axis_index("s")
        wid = s * NC + c
        pltpu.sync_copy(zeros_hbm, deg_sh.at[pl.ds(s * rpt, rpt)])
        pltpu.sync_copy(ones_hbm, srcv)
        pltpu.sync_copy(rows_hbm.at[wid], rowbuf)
        plsc.subcore_barrier()

        def body(j, carry):
            pltpu.sync_copy(srcv, deg_sh.at[rowbuf.at[j]], add=True)
            return carry

        lax.fori_loop(0, cpt, body, 0)
        plsc.subcore_barrier()
        pltpu.sync_copy(deg_sh.at[pl.ds(s * rpt, rpt)],
                        out_hbm.at[c, pl.ds(s * rpt, rpt)])

    return k


def _scatter_kernel(N, D, Ns, cpt, rpt):
    assert cpt % NBUF == 0 and (cpt * EC) % 128 == 0
    prows = cpt * EC // 128
    mesh = plsc.VectorSubcoreMesh(core_axis_name="c", subcore_axis_name="s",
                                  num_cores=NC, num_subcores=NS)

    scratch = [pltpu.VMEM((prows, 128), jnp.int32)]
    scratch += [pltpu.VMEM((EC,), jnp.int32) for _ in range(2 * NBUF)]
    scratch += [pltpu.VMEM((EC, D), jnp.float32) for _ in range(NBUF)]
    scratch += [pltpu.SemaphoreType.DMA for _ in range(NBUF)]
    scratch += [pltpu.VMEM_SHARED((Ns, D), jnp.float32)]

    @functools.partial(
        pl.kernel,
        out_type=jax.ShapeDtypeStruct((NC, Ns, D), jnp.float32),
        mesh=mesh,
        scratch_types=scratch,
    )
    def k(packed_hbm, g_hbm, zeros_hbm, out_hbm, pbuf, *rest):
        colvs = rest[0:NBUF]
        rowvs = rest[NBUF:2 * NBUF]
        gbufs = rest[2 * NBUF:3 * NBUF]
        sems = rest[3 * NBUF:4 * NBUF]
        s_sh = rest[4 * NBUF]
        c = lax.axis_index("c")
        s = lax.axis_index("s")
        wid = s * NC + c
        pltpu.sync_copy(zeros_hbm, s_sh.at[pl.ds(s * rpt, rpt)])
        pltpu.sync_copy(packed_hbm.at[wid], pbuf)
        plsc.subcore_barrier()

        def unpack(j, colv, rowv):
            for kk in range(EC // 16):
                off = j * EC + kk * 16
                v = pbuf[off // 128, pl.ds(off % 128, 16)]
                colv[pl.ds(kk * 16, 16)] = lax.bitwise_and(v, 16383)
                rowv[pl.ds(kk * 16, 16)] = lax.shift_right_logical(v, 14)

        for q in range(NBUF):
            unpack(q, colvs[q], rowvs[q])
            pltpu.async_copy(g_hbm.at[colvs[q]], gbufs[q], sems[q])

        def body(p, carry):
            for q in range(NBUF):
                j = NBUF * p + q
                pltpu.make_async_copy(g_hbm.at[colvs[q]], gbufs[q], sems[q]).wait()
                pltpu.sync_copy(gbufs[q], s_sh.at[rowvs[q]], add=True)

                @pl.when(j + NBUF < cpt)
                def _(q=q, j=j):
                    unpack(j + NBUF, colvs[q], rowvs[q])
                    pltpu.async_copy(g_hbm.at[colvs[q]], gbufs[q], sems[q])

            return carry

        lax.fori_loop(0, cpt // NBUF, body, 0)
        plsc.subcore_barrier()
        pltpu.sync_copy(s_sh.at[pl.ds(s * rpt, rpt)],
                        out_hbm.at[c, pl.ds(s * rpt, rpt)])

    return k


def _linear_kernel(x_ref, w_ref, b_ref, p0_ref, p1_ref, g_ref):
    d = p0_ref[:, 0:1] + p1_ref[:, 0:1] + 1.0
    dsq = lax.rsqrt(d)
    h = lax.dot_general(x_ref[...], w_ref[...],
                        (((1,), (1,)), ((), ())),
                        preferred_element_type=jnp.float32) + b_ref[...]
    g_ref[...] = h * dsq


def _final_kernel(g_ref, s0_ref, s1_ref, p0_ref, p1_ref, o_ref):
    d = p0_ref[:, 0:1] + p1_ref[:, 0:1] + 1.0
    dsq = lax.rsqrt(d)
    o_ref[...] = dsq * (g_ref[...] + s0_ref[...] + s1_ref[...])


def kernel(x, edge_index, W, b):
    N, Din = x.shape
    Dout = W.shape[0]
    E = edge_index.shape[1]
    NW = NC * NS

    rows = edge_index[0].astype(jnp.int32)
    cols = edge_index[1].astype(jnp.int32)
    Ns = 8 * NS * (-(-(N + 1) // (8 * NS)))
    rpt = Ns // NS
    cpt1 = -(-E // (CH * NW))
    pad1 = cpt1 * NW * CH - E
    rows_p = jnp.concatenate(
        [rows, jnp.full((pad1,), N, jnp.int32)]).reshape(cpt1, NW, CH).transpose(1, 0, 2)
    cpt = NBUF * (-(-E // (EC * NW * NBUF)))
    pad3 = cpt * NW * EC - E
    rows3 = jnp.concatenate([rows, jnp.full((pad3,), N, jnp.int32)])
    cols3 = jnp.concatenate([cols, jnp.zeros((pad3,), jnp.int32)])
    packed_p = (rows3 * 16384 + cols3).reshape(cpt, NW, EC).transpose(1, 0, 2) \
        .reshape(NW, cpt * EC // 128, 128)
    ones0 = jnp.zeros((CH, 128), jnp.float32).at[:, 0].set(1.0)
    zerosD = jnp.zeros((rpt, Dout), jnp.float32)
    b2 = b.reshape(1, Dout)

    degp = _deg_kernel(Ns, cpt1, rpt)(rows_p, ones0, zerosD)
    p0, p1 = degp[0], degp[1]

    NB = 400
    nblk = -(-N // NB)
    g = pl.pallas_call(
        _linear_kernel,
        grid=(nblk,),
        in_specs=[
            pl.BlockSpec((NB, Din), lambda i: (i, 0)),
            pl.BlockSpec((Dout, Din), lambda i: (0, 0)),
            pl.BlockSpec((1, Dout), lambda i: (0, 0)),
            pl.BlockSpec((NB, 128), lambda i: (i, 0)),
            pl.BlockSpec((NB, 128), lambda i: (i, 0)),
        ],
        out_specs=pl.BlockSpec((NB, Dout), lambda i: (i, 0)),
        out_shape=jax.ShapeDtypeStruct((N, Dout), jnp.float32),
    )(x, W, b2, p0, p1)

    sp = _scatter_kernel(N, Dout, Ns, cpt, rpt)(packed_p, g, zerosD)

    out = pl.pallas_call(
        _final_kernel,
        grid=(nblk,),
        in_specs=[
            pl.BlockSpec((NB, Dout), lambda i: (i, 0)),
            pl.BlockSpec((NB, Dout), lambda i: (i, 0)),
            pl.BlockSpec((NB, Dout), lambda i: (i, 0)),
            pl.BlockSpec((NB, 128), lambda i: (i, 0)),
            pl.BlockSpec((NB, 128), lambda i: (i, 0)),
        ],
        out_specs=pl.BlockSpec((NB, Dout), lambda i: (i, 0)),
        out_shape=jax.ShapeDtypeStruct((N, Dout), jnp.float32),
    )(g, sp[0], sp[1], p0, p1)
    return out


# speedup vs baseline: 16.4698x; 1.1922x over previous
import functools

import jax
import jax.numpy as jnp
from jax import lax
from jax.experimental import pallas as pl
from jax.experimental.pallas import tpu as pltpu, tpu_sc as plsc

NC = 2
NS = 16
CH = 128
EC = 64
NBUF = 4


def _deg_kernel(Nf, cpt):
    mesh = plsc.VectorSubcoreMesh(core_axis_name="c", subcore_axis_name="s",
                                  num_cores=NC, num_subcores=NS)

    @functools.partial(
        pl.kernel,
        out_type=jax.ShapeDtypeStruct((NC * NS, Nf), jnp.float32),
        mesh=mesh,
        scratch_types=[
            pltpu.VMEM((cpt, CH), jnp.int32),
            pltpu.VMEM((Nf,), jnp.float32),
        ],
        compiler_params=pltpu.CompilerParams(needs_layout_passes=False),
    )
    def k(rows_hbm, zeros_hbm, out_hbm, rowbuf, hist):
        c = lax.axis_index("c")
        s = lax.axis_index("s")
        wid = s * NC + c
        pltpu.sync_copy(zeros_hbm, hist)
        pltpu.sync_copy(rows_hbm.at[wid], rowbuf)
        ones = jnp.full((16,), 1.0, jnp.float32)

        def body(j, carry):
            for o in range(CH // 16):
                v = rowbuf[j, pl.ds(o * 16, 16)]
                plsc.addupdate_scatter(hist, [v], ones)
            return carry

        lax.fori_loop(0, cpt, body, 0)
        pltpu.sync_copy(hist, out_hbm.at[wid])

    return k


def _scatter_kernel(N, D, Ns, cpt, rpt):
    assert cpt % NBUF == 0 and (cpt * EC) % 128 == 0
    prows = cpt * EC // 128
    mesh = plsc.VectorSubcoreMesh(core_axis_name="c", subcore_axis_name="s",
                                  num_cores=NC, num_subcores=NS)

    scratch = [pltpu.VMEM((prows, 128), jnp.int32)]
    scratch += [pltpu.VMEM((EC,), jnp.int32) for _ in range(2 * NBUF)]
    scratch += [pltpu.VMEM((EC, D), jnp.float32) for _ in range(NBUF)]
    scratch += [pltpu.SemaphoreType.DMA for _ in range(NBUF)]
    scratch += [pltpu.VMEM_SHARED((Ns, D), jnp.float32)]

    @functools.partial(
        pl.kernel,
        out_type=jax.ShapeDtypeStruct((NC, Ns, D), jnp.float32),
        mesh=mesh,
        scratch_types=scratch,
    )
    def k(packed_hbm, g_hbm, zeros_hbm, out_hbm, pbuf, *rest):
        colvs = rest[0:NBUF]
        rowvs = rest[NBUF:2 * NBUF]
        gbufs = rest[2 * NBUF:3 * NBUF]
        sems = rest[3 * NBUF:4 * NBUF]
        s_sh = rest[4 * NBUF]
        c = lax.axis_index("c")
        s = lax.axis_index("s")
        wid = s * NC + c
        pltpu.sync_copy(zeros_hbm, s_sh.at[pl.ds(s * rpt, rpt)])
        pltpu.sync_copy(packed_hbm.at[wid], pbuf)
        plsc.subcore_barrier()

        def unpack(j, colv, rowv):
            for kk in range(EC // 16):
                off = j * EC + kk * 16
                v = pbuf[off // 128, pl.ds(off % 128, 16)]
                colv[pl.ds(kk * 16, 16)] = lax.bitwise_and(v, 16383)
                rowv[pl.ds(kk * 16, 16)] = lax.shift_right_logical(v, 14)

        for q in range(NBUF):
            unpack(q, colvs[q], rowvs[q])
            pltpu.async_copy(g_hbm.at[colvs[q]], gbufs[q], sems[q])

        def body(p, carry):
            for q in range(NBUF):
                j = NBUF * p + q
                pltpu.make_async_copy(g_hbm.at[colvs[q]], gbufs[q], sems[q]).wait()
                pltpu.sync_copy(gbufs[q], s_sh.at[rowvs[q]], add=True)

                @pl.when(j + NBUF < cpt)
                def _(q=q, j=j):
                    unpack(j + NBUF, colvs[q], rowvs[q])
                    pltpu.async_copy(g_hbm.at[colvs[q]], gbufs[q], sems[q])

            return carry

        lax.fori_loop(0, cpt // NBUF, body, 0)
        plsc.subcore_barrier()
        pltpu.sync_copy(s_sh.at[pl.ds(s * rpt, rpt)],
                        out_hbm.at[c, pl.ds(s * rpt, rpt)])

    return k


def _linear_kernel(x_ref, w_ref, b_ref, p_ref, g_ref):
    deg = jnp.sum(p_ref[...], axis=0) + 1.0
    dsq = lax.rsqrt(deg)[:, None]
    h = lax.dot_general(x_ref[...], w_ref[...],
                        (((1,), (1,)), ((), ())),
                        preferred_element_type=jnp.float32) + b_ref[...]
    g_ref[...] = h * dsq


def _final_kernel(g_ref, s0_ref, s1_ref, p_ref, o_ref):
    deg = jnp.sum(p_ref[...], axis=0) + 1.0
    dsq = lax.rsqrt(deg)[:, None]
    o_ref[...] = dsq * (g_ref[...] + s0_ref[0] + s1_ref[0])


def kernel(x, edge_index, W, b):
    N, Din = x.shape
    Dout = W.shape[0]
    E = edge_index.shape[1]
    NW = NC * NS

    rows = edge_index[0].astype(jnp.int32)
    cols = edge_index[1].astype(jnp.int32)
    Ns = 8 * NS * (-(-(N + 1) // (8 * NS)))
    rpt = Ns // NS
    cpt1 = -(-E // (CH * NW))
    pad1 = cpt1 * NW * CH - E
    rows_p = jnp.concatenate(
        [rows, jnp.full((pad1,), N, jnp.int32)]).reshape(cpt1, NW, CH).transpose(1, 0, 2)
    cpt = NBUF * (-(-E // (EC * NW * NBUF)))
    pad3 = cpt * NW * EC - E
    rows3 = jnp.concatenate([rows, jnp.full((pad3,), N, jnp.int32)])
    cols3 = jnp.concatenate([cols, jnp.zeros((pad3,), jnp.int32)])
    packed_p = (rows3 * 16384 + cols3).reshape(cpt, NW, EC).transpose(1, 0, 2) \
        .reshape(NW, cpt * EC // 128, 128)
    zerosD = jnp.zeros((rpt, Dout), jnp.float32)
    b2 = b.reshape(1, Dout)

    Nf = 128 * (-(-(N + 1) // 128))
    zerosF = jnp.zeros((Nf,), jnp.float32)
    degp = _deg_kernel(Nf, cpt1)(rows_p, zerosF)

    NB = 512
    nblk = -(-N // NB)
    g = pl.pallas_call(
        _linear_kernel,
        grid=(nblk,),
        in_specs=[
            pl.BlockSpec((NB, Din), lambda i: (i, 0)),
            pl.BlockSpec((Dout, Din), lambda i: (0, 0)),
            pl.BlockSpec((1, Dout), lambda i: (0, 0)),
            pl.BlockSpec((NW, NB), lambda i: (0, i)),
        ],
        out_specs=pl.BlockSpec((NB, Dout), lambda i: (i, 0)),
        out_shape=jax.ShapeDtypeStruct((N, Dout), jnp.float32),
    )(x, W, b2, degp)

    sp = _scatter_kernel(N, Dout, Ns, cpt, rpt)(packed_p, g, zerosD)

    out = pl.pallas_call(
        _final_kernel,
        grid=(nblk,),
        in_specs=[
            pl.BlockSpec((NB, Dout), lambda i: (i, 0)),
            pl.BlockSpec((1, NB, Dout), lambda i: (0, i, 0)),
            pl.BlockSpec((1, NB, Dout), lambda i: (1, i, 0)),
            pl.BlockSpec((NW, NB), lambda i: (0, i)),
        ],
        out_specs=pl.BlockSpec((NB, Dout), lambda i: (i, 0)),
        out_shape=jax.ShapeDtypeStruct((N, Dout), jnp.float32),
    )(g, sp, sp, degp)
    return out


# speedup vs baseline: 26.6183x; 1.6162x over previous
import functools

import jax
import jax.numpy as jnp
from jax import lax
from jax.experimental import pallas as pl
from jax.experimental.pallas import tpu as pltpu, tpu_sc as plsc

NC = 2
NS = 16
EC = 128


def _partition_kernel(Nf, prows, cap, half):
    NW = NC * NS
    mesh = plsc.VectorSubcoreMesh(core_axis_name="c", subcore_axis_name="s",
                                  num_cores=NC, num_subcores=NS)

    out_types = (
        jax.ShapeDtypeStruct((NW, Nf), jnp.float32),
        jax.ShapeDtypeStruct((NW, 4, cap), jnp.int32),
        jax.ShapeDtypeStruct((NW, 16), jnp.int32),
    )
    scratch = [pltpu.VMEM((prows, 128), jnp.int32),
               pltpu.VMEM((Nf,), jnp.float32)]
    scratch += [pltpu.VMEM((cap,), jnp.int32) for _ in range(4)]
    scratch += [pltpu.VMEM((16,), jnp.int32)]

    @functools.partial(
        pl.kernel, out_type=out_types, mesh=mesh, scratch_types=scratch,
        compiler_params=pltpu.CompilerParams(needs_layout_passes=False),
    )
    def k(packed_hbm, zeros_hbm, dummy_hbm, hist_out, lists_out, cnt_out,
          pbuf, hist, q0, q1, q2, q3, cbuf):
        c = lax.axis_index("c")
        s = lax.axis_index("s")
        wid = s * NC + c
        qbufs = (q0, q1, q2, q3)
        pltpu.sync_copy(zeros_hbm, hist)
        pltpu.sync_copy(packed_hbm.at[wid], pbuf)
        for q in range(4):
            pltpu.sync_copy(dummy_hbm.at[q], qbufs[q])
        ones = jnp.full((16,), 1.0, jnp.float32)

        def body(j, pos):
            p0, p1, p2, p3 = pos
            for o in range(128 // 16):
                v = pbuf[j, pl.ds(o * 16, 16)]
                row = lax.shift_right_logical(v, 14)
                col = lax.bitwise_and(v, 16383)
                plsc.addupdate_scatter(hist, [row], ones)
                rhi = row >= half
                chi = col >= half
                m0 = jnp.logical_and(jnp.logical_not(rhi), jnp.logical_not(chi))
                m1 = jnp.logical_and(jnp.logical_not(rhi), chi)
                m2 = jnp.logical_and(rhi, jnp.logical_not(chi))
                m3 = jnp.logical_and(rhi, chi)
                plsc.store_compressed(q0.at[pl.ds(p0, 16)], v, mask=m0)
                plsc.store_compressed(q1.at[pl.ds(p1, 16)], v, mask=m1)
                plsc.store_compressed(q2.at[pl.ds(p2, 16)], v, mask=m2)
                plsc.store_compressed(q3.at[pl.ds(p3, 16)], v, mask=m3)
                p0 = p0 + jnp.sum(m0.astype(jnp.int32))
                p1 = p1 + jnp.sum(m1.astype(jnp.int32))
                p2 = p2 + jnp.sum(m2.astype(jnp.int32))
                p3 = p3 + jnp.sum(m3.astype(jnp.int32))
            return (p0, p1, p2, p3)

        z = jnp.int32(0)
        pos = lax.fori_loop(0, prows, body, (z, z, z, z))
        for q in range(4):
            pltpu.sync_copy(qbufs[q], lists_out.at[wid, q])
        io16 = lax.iota(jnp.int32, 16)
        cv = jnp.where(io16 == 0, pos[0],
             jnp.where(io16 == 1, pos[1],
             jnp.where(io16 == 2, pos[2],
             jnp.where(io16 == 3, pos[3], 0))))
        cbuf[...] = cv
        pltpu.sync_copy(cbuf, cnt_out.at[wid])
        pltpu.sync_copy(hist, hist_out.at[wid])

    return k


def _aggr_kernel(D, cap, half, HR):
    rp = HR // NS
    rg = half // NS
    mesh = plsc.VectorSubcoreMesh(core_axis_name="c", subcore_axis_name="s",
                                  num_cores=NC, num_subcores=NS)

    scratch = [
        pltpu.VMEM((EC,), jnp.int32),
        pltpu.VMEM((EC,), jnp.int32),
        pltpu.VMEM((EC,), jnp.int32),
        pltpu.VMEM((EC, D), jnp.float32),
        pltpu.VMEM((16,), jnp.int32),
        pltpu.VMEM((16,), jnp.int32),
        pltpu.SemaphoreType.DMA,
        pltpu.VMEM_SHARED((half, D), jnp.float32),
        pltpu.VMEM_SHARED((HR, D), jnp.float32),
    ]

    @functools.partial(
        pl.kernel,
        out_type=jax.ShapeDtypeStruct((NC, HR, D), jnp.float32),
        mesh=mesh, scratch_types=scratch,
        compiler_params=pltpu.CompilerParams(needs_layout_passes=False),
    )
    def k(lists_hbm, cnts_hbm, g_hbm, zeros_hbm, out_hbm,
          lbuf, colv, rowv, gbuf, cb0, cb1, sem, g_sh, acc):
        c = lax.axis_index("c")
        s = lax.axis_index("s")
        pltpu.sync_copy(zeros_hbm, acc.at[pl.ds(s * rp, rp)])
        pltpu.sync_copy(cnts_hbm.at[2 * s], cb0)
        pltpu.sync_copy(cnts_hbm.at[2 * s + 1], cb1)
        io16 = lax.iota(jnp.int32, 16)
        cv0 = cb0[...]
        cv1 = cb1[...]

        def stage(ch):
            pltpu.sync_copy(g_hbm.at[pl.ds(ch * half + s * rg, rg)],
                            g_sh.at[pl.ds(s * rg, rg)])

        def process(k0t, q, cnt, col_base):
            nch = lax.div(cnt + (EC - 1), jnp.int32(EC))

            def body(j, carry):
                pltpu.sync_copy(lists_hbm.at[k0t, q, pl.ds(j * EC, EC)], lbuf)
                for o in range(EC // 16):
                    v = lbuf[pl.ds(o * 16, 16)]
                    colv[pl.ds(o * 16, 16)] = \
                        lax.bitwise_and(v, 16383) - col_base
                    rowv[pl.ds(o * 16, 16)] = \
                        lax.shift_right_logical(v, 14) - c * half
                pltpu.async_copy(g_sh.at[colv], gbuf, sem).wait()
                pltpu.sync_copy(gbuf, acc.at[rowv], add=True)
                return carry

            lax.fori_loop(0, nch, body, 0)

        def cnt_of(cv, q):
            return jnp.sum(jnp.where(io16 == q, cv, 0))

        stage(c)
        plsc.subcore_barrier()
        qa = 3 * c
        process(2 * s, qa, cnt_of(cv0, qa), c * half)
        process(2 * s + 1, qa, cnt_of(cv1, qa), c * half)
        plsc.subcore_barrier()
        stage(1 - c)
        plsc.subcore_barrier()
        qb = 2 * c + (1 - c)
        process(2 * s, qb, cnt_of(cv0, qb), (1 - c) * half)
        process(2 * s + 1, qb, cnt_of(cv1, qb), (1 - c) * half)
        plsc.subcore_barrier()
        pltpu.sync_copy(acc.at[pl.ds(s * rp, rp)],
                        out_hbm.at[c, pl.ds(s * rp, rp)])

    return k


def _linear_kernel(x_ref, w_ref, b_ref, p_ref, g_ref):
    deg = jnp.sum(p_ref[...], axis=0) + 1.0
    dsq = lax.rsqrt(deg)[:, None]
    h = lax.dot_general(x_ref[...], w_ref[...],
                        (((1,), (1,)), ((), ())),
                        preferred_element_type=jnp.float32) + b_ref[...]
    g_ref[...] = h * dsq


def _final_kernel(g_ref, s_ref, p_ref, o_ref):
    deg = jnp.sum(p_ref[...], axis=0) + 1.0
    dsq = lax.rsqrt(deg)[:, None]
    o_ref[...] = dsq * (g_ref[...] + s_ref[0])


def kernel(x, edge_index, W, b):
    N, Din = x.shape
    Dout = W.shape[0]
    E = edge_index.shape[1]
    NW = NC * NS
    NB = 512
    half = NB * (-(-N // (2 * NB)))
    HR = half + NB

    rows = edge_index[0].astype(jnp.int32)
    cols = edge_index[1].astype(jnp.int32)
    ept = 128 * (-(-E // (128 * NW)))
    prows = ept // 128
    cap = ept
    pad = ept * NW - E
    rows_pad = jnp.full((pad,), N, jnp.int32)
    cols_pad = jnp.zeros((pad,), jnp.int32)
    packed = (jnp.concatenate([rows, rows_pad]) * 16384
              + jnp.concatenate([cols, cols_pad]))
    packed_p = packed.reshape(prows, NW, 128).transpose(1, 0, 2)
    dummies = jnp.array(
        [half * 16384 + 0, half * 16384 + half,
         N * 16384 + 0, N * 16384 + half], jnp.int32)
    dummy_hbm = jnp.broadcast_to(dummies[:, None], (4, cap))
    Nf = 128 * (-(-(N + 1) // 128))
    zerosF = jnp.zeros((Nf,), jnp.float32)
    zerosR = jnp.zeros((HR // NS, Dout), jnp.float32)
    assert half % NB == 0 and HR % (16 * NS) == 0 and half % (16 * NS) == 0
    b2 = b.reshape(1, Dout)

    hists, lists, cnts = _partition_kernel(Nf, prows, cap, half)(
        packed_p, zerosF, dummy_hbm)

    nblk = -(-N // NB)
    Ng = NB * nblk
    g = pl.pallas_call(
        _linear_kernel,
        grid=(nblk,),
        in_specs=[
            pl.BlockSpec((NB, Din), lambda i: (i, 0)),
            pl.BlockSpec((Dout, Din), lambda i: (0, 0)),
            pl.BlockSpec((1, Dout), lambda i: (0, 0)),
            pl.BlockSpec((NW, NB), lambda i: (0, i)),
        ],
        out_specs=pl.BlockSpec((NB, Dout), lambda i: (i, 0)),
        out_shape=jax.ShapeDtypeStruct((Ng, Dout), jnp.float32),
    )(x, W, b2, hists)

    sp = _aggr_kernel(Dout, cap, half, HR)(lists, cnts, g, zerosR)

    blocks_per_half = half // NB
    out = pl.pallas_call(
        _final_kernel,
        grid=(nblk,),
        in_specs=[
            pl.BlockSpec((NB, Dout), lambda i: (i, 0)),
            pl.BlockSpec((1, NB, Dout),
                         lambda i: (i // blocks_per_half,
                                    i % blocks_per_half, 0)),
            pl.BlockSpec((NW, NB), lambda i: (0, i)),
        ],
        out_specs=pl.BlockSpec((NB, Dout), lambda i: (i, 0)),
        out_shape=jax.ShapeDtypeStruct((N, Dout), jnp.float32),
    )(g, sp, hists)
    return out


# speedup vs baseline: 26.8372x; 1.0082x over previous
import functools

import jax
import jax.numpy as jnp
from jax import lax
from jax.experimental import pallas as pl
from jax.experimental.pallas import tpu as pltpu, tpu_sc as plsc

NC = 2
NS = 16
EC = 128


def _partition_kernel(Nf, prows, cap, half, nreal):
    NW = NC * NS
    mesh = plsc.VectorSubcoreMesh(core_axis_name="c", subcore_axis_name="s",
                                  num_cores=NC, num_subcores=NS)

    out_types = (
        jax.ShapeDtypeStruct((NW, Nf), jnp.float32),
        jax.ShapeDtypeStruct((NW, 4, cap), jnp.int32),
        jax.ShapeDtypeStruct((NW, 16), jnp.int32),
    )
    scratch = [pltpu.VMEM((prows, 128), jnp.int32),
               pltpu.VMEM((Nf,), jnp.float32)]
    scratch += [pltpu.VMEM((cap,), jnp.int32) for _ in range(4)]
    scratch += [pltpu.VMEM((16,), jnp.int32)]

    @functools.partial(
        pl.kernel, out_type=out_types, mesh=mesh, scratch_types=scratch,
        compiler_params=pltpu.CompilerParams(needs_layout_passes=False),
    )
    def k(packed_hbm, zeros_hbm, dummy_hbm, hist_out, lists_out, cnt_out,
          pbuf, hist, q0, q1, q2, q3, cbuf):
        c = lax.axis_index("c")
        s = lax.axis_index("s")
        wid = s * NC + c
        qbufs = (q0, q1, q2, q3)
        pltpu.sync_copy(zeros_hbm, hist)
        pltpu.sync_copy(packed_hbm.at[wid], pbuf)
        for q in range(4):
            pltpu.sync_copy(dummy_hbm.at[q], qbufs[q])
        ones = jnp.full((16,), 1.0, jnp.float32)

        def body(j, pos):
            p0, p1, p2, p3 = pos
            for o in range(128 // 16):
                v = pbuf[j, pl.ds(o * 16, 16)]
                row = lax.shift_right_logical(v, 14)
                col = lax.bitwise_and(v, 16383)
                plsc.addupdate_scatter(hist, [row], ones)
                rhi = row >= half
                chi = col >= half
                rlo = row < half
                val = jnp.logical_and(rhi, row < nreal)
                m0 = jnp.logical_and(rlo, jnp.logical_not(chi))
                m1 = jnp.logical_and(rlo, chi)
                m2 = jnp.logical_and(val, jnp.logical_not(chi))
                m3 = jnp.logical_and(val, chi)
                plsc.store_compressed(q0.at[pl.ds(p0, 16)], v, mask=m0)
                plsc.store_compressed(q1.at[pl.ds(p1, 16)], v, mask=m1)
                plsc.store_compressed(q2.at[pl.ds(p2, 16)], v, mask=m2)
                plsc.store_compressed(q3.at[pl.ds(p3, 16)], v, mask=m3)
                p0 = p0 + jnp.sum(m0.astype(jnp.int32))
                p1 = p1 + jnp.sum(m1.astype(jnp.int32))
                p2 = p2 + jnp.sum(m2.astype(jnp.int32))
                p3 = p3 + jnp.sum(m3.astype(jnp.int32))
            return (p0, p1, p2, p3)

        z = jnp.int32(0)
        pos = lax.fori_loop(0, prows, body, (z, z, z, z))
        for q in range(4):
            pltpu.sync_copy(qbufs[q], lists_out.at[wid, q])
        io16 = lax.iota(jnp.int32, 16)
        cv = jnp.where(io16 == 0, pos[0],
             jnp.where(io16 == 1, pos[1],
             jnp.where(io16 == 2, pos[2],
             jnp.where(io16 == 3, pos[3], 0))))
        cbuf[...] = cv
        pltpu.sync_copy(cbuf, cnt_out.at[wid])
        pltpu.sync_copy(hist, hist_out.at[wid])

    return k


def _aggr_kernel(D, cap, half, HR):
    rp = HR // NS
    rg = half // NS
    mesh = plsc.VectorSubcoreMesh(core_axis_name="c", subcore_axis_name="s",
                                  num_cores=NC, num_subcores=NS)

    scratch = [
        pltpu.VMEM((EC,), jnp.int32),
        pltpu.VMEM((EC,), jnp.int32),
        pltpu.VMEM((EC,), jnp.int32),
        pltpu.VMEM((EC, D), jnp.float32),
        pltpu.VMEM((16,), jnp.int32),
        pltpu.VMEM((16,), jnp.int32),
        pltpu.SemaphoreType.DMA,
        pltpu.VMEM_SHARED((half, D), jnp.float32),
        pltpu.VMEM_SHARED((HR, D), jnp.float32),
    ]

    @functools.partial(
        pl.kernel,
        out_type=jax.ShapeDtypeStruct((NC, HR, D), jnp.float32),
        mesh=mesh, scratch_types=scratch,
        compiler_params=pltpu.CompilerParams(needs_layout_passes=False),
    )
    def k(lists_hbm, cnts_hbm, g_hbm, zeros_hbm, out_hbm,
          lbuf, colv, rowv, gbuf, cb0, cb1, sem, g_sh, acc):
        c = lax.axis_index("c")
        s = lax.axis_index("s")
        pltpu.sync_copy(zeros_hbm, acc.at[pl.ds(s * rp, rp)])
        pltpu.sync_copy(cnts_hbm.at[2 * s], cb0)
        pltpu.sync_copy(cnts_hbm.at[2 * s + 1], cb1)
        io16 = lax.iota(jnp.int32, 16)
        cv0 = cb0[...]
        cv1 = cb1[...]

        def stage(ch):
            pltpu.sync_copy(g_hbm.at[pl.ds(ch * half + s * rg, rg)],
                            g_sh.at[pl.ds(s * rg, rg)])

        def process(k0t, q, cnt, col_base):
            nch = lax.div(cnt + (EC - 1), jnp.int32(EC))

            def body(j, carry):
                pltpu.sync_copy(lists_hbm.at[k0t, q, pl.ds(j * EC, EC)], lbuf)
                for o in range(EC // 16):
                    v = lbuf[pl.ds(o * 16, 16)]
                    colv[pl.ds(o * 16, 16)] = \
                        lax.bitwise_and(v, 16383) - col_base
                    rowv[pl.ds(o * 16, 16)] = \
                        lax.shift_right_logical(v, 14) - c * half
                pltpu.async_copy(g_sh.at[colv], gbuf, sem).wait()
                pltpu.sync_copy(gbuf, acc.at[rowv], add=True)
                return carry

            lax.fori_loop(0, nch, body, 0)

        def cnt_of(cv, q):
            return jnp.sum(jnp.where(io16 == q, cv, 0))

        stage(c)
        plsc.subcore_barrier()
        qa = 3 * c
        process(2 * s, qa, cnt_of(cv0, qa), c * half)
        process(2 * s + 1, qa, cnt_of(cv1, qa), c * half)
        plsc.subcore_barrier()
        stage(1 - c)
        plsc.subcore_barrier()
        qb = 2 * c + (1 - c)
        process(2 * s, qb, cnt_of(cv0, qb), (1 - c) * half)
        process(2 * s + 1, qb, cnt_of(cv1, qb), (1 - c) * half)
        plsc.subcore_barrier()
        pltpu.sync_copy(acc.at[pl.ds(s * rp, rp)],
                        out_hbm.at[c, pl.ds(s * rp, rp)])

    return k


def _linear_kernel(x_ref, w_ref, b_ref, p_ref, g_ref):
    deg = jnp.sum(p_ref[...], axis=0) + 1.0
    dsq = lax.rsqrt(deg)[:, None]
    h = lax.dot_general(x_ref[...], w_ref[...],
                        (((1,), (1,)), ((), ())),
                        preferred_element_type=jnp.float32) + b_ref[...]
    g_ref[...] = h * dsq


def _final_kernel(g_ref, s_ref, p_ref, o_ref):
    deg = jnp.sum(p_ref[...], axis=0) + 1.0
    dsq = lax.rsqrt(deg)[:, None]
    o_ref[...] = dsq * (g_ref[...] + s_ref[0])


def kernel(x, edge_index, W, b):
    N, Din = x.shape
    Dout = W.shape[0]
    E = edge_index.shape[1]
    NW = NC * NS
    NB = 512
    half = NB * (-(-N // (2 * NB)))
    HR = half + NB

    rows = edge_index[0].astype(jnp.int32)
    cols = edge_index[1].astype(jnp.int32)
    ept = 128 * (-(-E // (128 * NW)))
    prows = ept // 128
    cap = ept
    pad = ept * NW - E
    rows_pad = jnp.full((pad,), N, jnp.int32)
    cols_pad = jnp.zeros((pad,), jnp.int32)
    packed = (jnp.concatenate([rows, rows_pad]) * 16384
              + jnp.concatenate([cols, cols_pad]))
    packed_p = packed.reshape(NW, prows, 128)
    dummies = jnp.array(
        [half * 16384 + 0, half * 16384 + half,
         N * 16384 + 0, N * 16384 + half], jnp.int32)
    dummy_hbm = jnp.broadcast_to(dummies[:, None], (4, cap))
    Nf = 128 * (-(-(N + 1) // 128))
    zerosF = jnp.zeros((Nf,), jnp.float32)
    zerosR = jnp.zeros((HR // NS, Dout), jnp.float32)
    assert half % NB == 0 and HR % (16 * NS) == 0 and half % (16 * NS) == 0
    b2 = b.reshape(1, Dout)

    hists, lists, cnts = _partition_kernel(Nf, prows, cap, half, N)(
        packed_p, zerosF, dummy_hbm)

    nblk = -(-N // NB)
    Ng = NB * nblk
    g = pl.pallas_call(
        _linear_kernel,
        grid=(nblk,),
        in_specs=[
            pl.BlockSpec((NB, Din), lambda i: (i, 0)),
            pl.BlockSpec((Dout, Din), lambda i: (0, 0)),
            pl.BlockSpec((1, Dout), lambda i: (0, 0)),
            pl.BlockSpec((NW, NB), lambda i: (0, i)),
        ],
        out_specs=pl.BlockSpec((NB, Dout), lambda i: (i, 0)),
        out_shape=jax.ShapeDtypeStruct((Ng, Dout), jnp.float32),
    )(x, W, b2, hists)

    sp = _aggr_kernel(Dout, cap, half, HR)(lists, cnts, g, zerosR)

    blocks_per_half = half // NB
    out = pl.pallas_call(
        _final_kernel,
        grid=(nblk,),
        in_specs=[
            pl.BlockSpec((NB, Dout), lambda i: (i, 0)),
            pl.BlockSpec((1, NB, Dout),
                         lambda i: (i // blocks_per_half,
                                    i % blocks_per_half, 0)),
            pl.BlockSpec((NW, NB), lambda i: (0, i)),
        ],
        out_specs=pl.BlockSpec((NB, Dout), lambda i: (i, 0)),
        out_shape=jax.ShapeDtypeStruct((N, Dout), jnp.float32),
    )(g, sp, hists)
    return out
